# baseline (speedup 1.0000x reference)
import functools

import jax
import jax.numpy as jnp
from jax import lax
from jax.experimental import pallas as pl
from jax.experimental.pallas import tpu as pltpu
from jax.experimental.pallas import tpu_sc as plsc

N_NODES = 10000
D = 128
NPAD = 10240
DUMMY = NPAD - 1
NCORES = 2
NSUB = 16
NW = NCORES * NSUB
NDEV = 2
K = 64
NBUF = 4
E_TOT = 320000 + N_NODES
E_DEV = E_TOT // NDEV
G = 6
NGA = 8
NGB = 20
NGMAX = max(NGA, NGB)
CAP_A = NGA * G * K
CAP_B = NGB * G * K
EPAD = NSUB * (CAP_A + CAP_B)
assert EPAD >= E_DEV and NGA % 2 == 0 and NGB % 2 == 0
assert NBUF - 1 < G and (2 * G) % NBUF == 0
ROWS_PER_SUB = NPAD // NSUB
DW = 128
RB = 512

_vector_mesh = plsc.VectorSubcoreMesh(core_axis_name="c", subcore_axis_name="s",
                                      num_cores=NCORES, num_subcores=NSUB)


def _fill(buf, rows, width, value):
    v16 = jnp.full((16,), value, jnp.float32)

    @pl.loop(0, rows)
    def _(r):
        row = buf.at[r]
        for t in range(width // 16):
            row[pl.ds(t * 16, 16)] = v16


def _deg_body(dst_hbm, out_hbm, acc, didx, ones_buf, zero_buf):
    c = lax.axis_index("c")
    s = lax.axis_index("s")
    wid = c * NSUB + s

    _fill(ones_buf, K, DW, 1.0)
    _fill(zero_buf, K, DW, 0.0)
    for t in range(ROWS_PER_SUB // K):
        pltpu.sync_copy(zero_buf, acc.at[pl.ds(s * ROWS_PER_SUB + t * K, K)])
    plsc.subcore_barrier()
    ng_self = jnp.where(c == 0, NGA, NGB)

    @pl.loop(0, ng_self)
    def _(jg):
        pltpu.sync_copy(dst_hbm.at[wid, jg], didx)
        for b in range(G):
            pltpu.sync_copy(ones_buf, acc.at[didx.at[b]], add=True)

    plsc.subcore_barrier()
    pltpu.sync_copy(acc.at[pl.ds(s * ROWS_PER_SUB, ROWS_PER_SUB)],
                    out_hbm.at[c, pl.ds(s * ROWS_PER_SUB, ROWS_PER_SUB)])


def _segsum_body(g_hbm, src_hbm, dst_hbm, out_hbm, acc, s0, d0, s1, d1, *rest):
    bufs = rest[:NBUF]
    sems = rest[NBUF:2 * NBUF]
    semI = rest[2 * NBUF]
    c = lax.axis_index("c")
    s = lax.axis_index("s")
    wid = c * NSUB + s
    P = NBUF - 1

    def _wait_gather(i):
        pltpu.make_async_copy(g_hbm.at[s0.at[0]], bufs[i], sems[i]).wait()

    _fill(bufs[0], K, D, 0.0)
    for t in range(ROWS_PER_SUB // K):
        pltpu.sync_copy(bufs[0], acc.at[pl.ds(s * ROWS_PER_SUB + t * K, K)])
    plsc.subcore_barrier()

    pltpu.sync_copy(src_hbm.at[wid, 0], s0)
    pltpu.sync_copy(dst_hbm.at[wid, 0], d0)
    for i in range(P):
        pltpu.async_copy(g_hbm.at[s0.at[i]], bufs[i], sems[i])

    def _group(jg, si, di, sn, dn, base):
        ia = pltpu.async_copy(src_hbm.at[wid, jg + 1], sn, semI)
        ib = pltpu.async_copy(dst_hbm.at[wid, jg + 1], dn, semI)
        waited_idx = False
        for b in range(G):
            bp = b + P
            if bp < G:
                src_ref = si.at[bp]
            else:
                if not waited_idx:
                    ia.wait()
                    ib.wait()
                    waited_idx = True
                src_ref = sn.at[bp - G]
            slot_issue = (base + bp) % NBUF
            pltpu.async_copy(g_hbm.at[src_ref], bufs[slot_issue], sems[slot_issue])
            slot = (base + b) % NBUF
            _wait_gather(slot)
            pltpu.sync_copy(bufs[slot], acc.at[di.at[b]], add=True)

    ng_self = jnp.where(c == 0, NGA, NGB)

    @pl.loop(0, ng_self, step=2)
    def _(jg):
        _group(jg, s0, d0, s1, d1, 0)
        _group(jg + 1, s1, d1, s0, d0, G % NBUF)

    for i in range(P):
        _wait_gather(i % NBUF)
    plsc.subcore_barrier()
    pltpu.sync_copy(acc.at[pl.ds(s * ROWS_PER_SUB, ROWS_PER_SUB)],
                    out_hbm.at[c, pl.ds(s * ROWS_PER_SUB, ROWS_PER_SUB)])


def _make_deg_kernel(interpret=False):
    return pl.kernel(
        _deg_body,
        out_type=jax.ShapeDtypeStruct((NCORES, NPAD, DW), jnp.float32),
        mesh=_vector_mesh,
        scratch_types=[
            pltpu.VMEM_SHARED((NPAD, DW), jnp.float32),
            pltpu.VMEM((G, K), jnp.int32),
            pltpu.VMEM((K, DW), jnp.float32),
            pltpu.VMEM((K, DW), jnp.float32),
        ],
        interpret=interpret,
    )


def _make_segsum_kernel(interpret=False):
    return pl.kernel(
        _segsum_body,
        out_type=jax.ShapeDtypeStruct((NCORES, NPAD, D), jnp.float32),
        mesh=_vector_mesh,
        scratch_types=(
            [pltpu.VMEM_SHARED((NPAD, D), jnp.float32)]
            + [pltpu.VMEM((G, K), jnp.int32)] * 4
            + [pltpu.VMEM((K, D), jnp.float32)] * NBUF
            + [pltpu.SemaphoreType.DMA] * (NBUF + 1)
        ),
        interpret=interpret,
    )


_deg_kernel = _make_deg_kernel()
_segsum_kernel = _make_segsum_kernel()


def _dinv_block(dcol):
    return jnp.where(dcol > 0.0, lax.rsqrt(jnp.maximum(dcol, 1e-30)), 0.0)


def _mm_first_body(x_ref, w_ref, dcol_ref, g_ref):
    dv = _dinv_block(dcol_ref[...])
    h = jnp.dot(x_ref[...], w_ref[...], preferred_element_type=jnp.float32)
    g_ref[...] = h * dv


def _mm_mid_body(accs_ref, dcol_ref, b_ref, w_ref, g_ref):
    dv = _dinv_block(dcol_ref[...])
    act = jnp.maximum(dv * (accs_ref[0] + accs_ref[1]) + b_ref[...], 0.0)
    g_ref[...] = jnp.dot(act, w_ref[...], preferred_element_type=jnp.float32) * dv


def _epilogue_body(accs_ref, dcol_ref, b_ref, out_ref):
    dv = _dinv_block(dcol_ref[...])
    out_ref[...] = jnp.maximum(dv * (accs_ref[0] + accs_ref[1]) + b_ref[...], 0.0)


_GRID = (NPAD // RB,)
_accs_spec = pl.BlockSpec((NCORES, RB, D), lambda i: (0, i, 0))
_dcol_spec = pl.BlockSpec((RB, 1), lambda i: (i, 0))
_b_spec = pl.BlockSpec((1, D), lambda i: (0, 0))
_w_spec = pl.BlockSpec((D, D), lambda i: (0, 0))
_row_spec = pl.BlockSpec((RB, D), lambda i: (i, 0))

_mm_first = pl.pallas_call(
    _mm_first_body,
    grid=_GRID,
    in_specs=[_row_spec, _w_spec, _dcol_spec],
    out_specs=_row_spec,
    out_shape=jax.ShapeDtypeStruct((NPAD, D), jnp.float32),
)

_mm_mid = pl.pallas_call(
    _mm_mid_body,
    grid=_GRID,
    in_specs=[_accs_spec, _dcol_spec, _b_spec, _w_spec],
    out_specs=_row_spec,
    out_shape=jax.ShapeDtypeStruct((NPAD, D), jnp.float32),
)

_epilogue = pl.pallas_call(
    _epilogue_body,
    grid=_GRID,
    in_specs=[_accs_spec, _dcol_spec, _b_spec],
    out_specs=_row_spec,
    out_shape=jax.ShapeDtypeStruct((NPAD, D), jnp.float32),
)


def _pipeline(src2d, dst2d, xp, W1, b1r, W2, b2r, W3, b3r):
    src2d = src2d[0]
    dst2d = dst2d[0]
    deg_parts = _deg_kernel(dst2d)
    dcol = lax.psum(deg_parts[0, :, 0:1] + deg_parts[1, :, 0:1], "dev")

    g = _mm_first(xp, W1, dcol)
    accs = lax.psum(_segsum_kernel(g, src2d, dst2d), "dev")
    g = _mm_mid(accs, dcol, b1r, W2)
    accs = lax.psum(_segsum_kernel(g, src2d, dst2d), "dev")
    g = _mm_mid(accs, dcol, b2r, W3)
    accs = lax.psum(_segsum_kernel(g, src2d, dst2d), "dev")
    return _epilogue(accs, dcol, b3r)


def kernel(x, edge_index, W1, b1, W2, b2, W3, b3):
    loop = jnp.arange(N_NODES, dtype=jnp.int32)
    src = jnp.concatenate([edge_index[0].astype(jnp.int32), loop])
    dst = jnp.concatenate([edge_index[1].astype(jnp.int32), loop])
    pad = jnp.full((EPAD - E_DEV,), DUMMY, dtype=jnp.int32)

    def build(v):
        parts = []
        for d in range(NDEV):
            vd = jnp.concatenate([v[d * E_DEV:(d + 1) * E_DEV], pad])
            ea = vd[:NSUB * CAP_A].reshape(NSUB, NGA, G, K)
            eb = vd[NSUB * CAP_A:].reshape(NSUB, NGB, G, K)
            da = jnp.full((NSUB, NGMAX + 1 - NGA, G, K), DUMMY, jnp.int32)
            db = jnp.full((NSUB, NGMAX + 1 - NGB, G, K), DUMMY, jnp.int32)
            parts.append(jnp.concatenate([
                jnp.concatenate([ea, da], axis=1),
                jnp.concatenate([eb, db], axis=1)], axis=0))
        return jnp.stack(parts)

    src2d = build(src)
    dst2d = build(dst)

    xp = jnp.pad(x, ((0, NPAD - N_NODES), (0, 0)))
    b1r = b1.reshape(1, D)
    b2r = b2.reshape(1, D)
    b3r = b3.reshape(1, D)

    import numpy as _np
    from jax.sharding import Mesh, PartitionSpec as PS
    mesh = Mesh(_np.asarray(jax.devices()[:NDEV]), ("dev",))
    rep = PS()
    shfn = jax.shard_map(
        _pipeline, mesh=mesh,
        in_specs=(PS("dev"), PS("dev"), rep, rep, rep, rep, rep, rep, rep),
        out_specs=rep, check_vma=False)
    out = shfn(src2d, dst2d, xp, W1, b1r, W2, b2r, W3, b3r)
    return out[:N_NODES]


# speedup vs baseline: 1.0173x; 1.0173x over previous
import functools

import jax
import jax.numpy as jnp
from jax import lax
from jax.experimental import pallas as pl
from jax.experimental.pallas import tpu as pltpu
from jax.experimental.pallas import tpu_sc as plsc

N_NODES = 10000
D = 128
NPAD = 10240
DUMMY = NPAD - 1
NCORES = 2
NSUB = 16
NW = NCORES * NSUB
NDEV = 2
K = 64
NBUF = 4
E_TOT = 320000 + N_NODES
E_DEV = E_TOT // NDEV
G = 6
NGA = 20
NGB = 8
NGMAX = max(NGA, NGB)
CAP_A = NGA * G * K
CAP_B = NGB * G * K
EPAD = NSUB * (CAP_A + CAP_B)
assert EPAD >= E_DEV and NGA % 2 == 0 and NGB % 2 == 0
assert NBUF - 1 < G and (2 * G) % NBUF == 0
ROWS_PER_SUB = NPAD // NSUB
DW = 128
RB = 512

_vector_mesh = plsc.VectorSubcoreMesh(core_axis_name="c", subcore_axis_name="s",
                                      num_cores=NCORES, num_subcores=NSUB)


def _fill(buf, rows, width, value):
    v16 = jnp.full((16,), value, jnp.float32)

    @pl.loop(0, rows)
    def _(r):
        row = buf.at[r]
        for t in range(width // 16):
            row[pl.ds(t * 16, 16)] = v16


def _deg_body(dst_hbm, out_hbm, acc, didx, ones_buf, zero_buf):
    c = lax.axis_index("c")
    s = lax.axis_index("s")
    wid = c * NSUB + s

    _fill(ones_buf, K, DW, 1.0)
    _fill(zero_buf, K, DW, 0.0)
    for t in range(ROWS_PER_SUB // K):
        pltpu.sync_copy(zero_buf, acc.at[pl.ds(s * ROWS_PER_SUB + t * K, K)])
    plsc.subcore_barrier()
    ng_self = jnp.where(c == 0, NGA, NGB)

    @pl.loop(0, ng_self)
    def _(jg):
        pltpu.sync_copy(dst_hbm.at[wid, jg], didx)
        for b in range(G):
            pltpu.sync_copy(ones_buf, acc.at[didx.at[b]], add=True)

    plsc.subcore_barrier()
    pltpu.sync_copy(acc.at[pl.ds(s * ROWS_PER_SUB, ROWS_PER_SUB)],
                    out_hbm.at[c, pl.ds(s * ROWS_PER_SUB, ROWS_PER_SUB)])


def _segsum_body(g_hbm, src_hbm, dst_hbm, out_hbm, acc, s0, d0, s1, d1, *rest):
    bufs = rest[:NBUF]
    sems = rest[NBUF:2 * NBUF]
    semI = rest[2 * NBUF]
    c = lax.axis_index("c")
    s = lax.axis_index("s")
    wid = c * NSUB + s
    P = NBUF - 1

    def _wait_gather(i):
        pltpu.make_async_copy(g_hbm.at[s0.at[0]], bufs[i], sems[i]).wait()

    _fill(bufs[0], K, D, 0.0)
    for t in range(ROWS_PER_SUB // K):
        pltpu.sync_copy(bufs[0], acc.at[pl.ds(s * ROWS_PER_SUB + t * K, K)])
    plsc.subcore_barrier()

    pltpu.sync_copy(src_hbm.at[wid, 0], s0)
    pltpu.sync_copy(dst_hbm.at[wid, 0], d0)
    for i in range(P):
        pltpu.async_copy(g_hbm.at[s0.at[i]], bufs[i], sems[i])

    def _group(jg, si, di, sn, dn, base):
        ia = pltpu.async_copy(src_hbm.at[wid, jg + 1], sn, semI)
        ib = pltpu.async_copy(dst_hbm.at[wid, jg + 1], dn, semI)
        waited_idx = False
        for b in range(G):
            bp = b + P
            if bp < G:
                src_ref = si.at[bp]
            else:
                if not waited_idx:
                    ia.wait()
                    ib.wait()
                    waited_idx = True
                src_ref = sn.at[bp - G]
            slot_issue = (base + bp) % NBUF
            pltpu.async_copy(g_hbm.at[src_ref], bufs[slot_issue], sems[slot_issue])
            slot = (base + b) % NBUF
            _wait_gather(slot)
            pltpu.sync_copy(bufs[slot], acc.at[di.at[b]], add=True)

    ng_self = jnp.where(c == 0, NGA, NGB)

    @pl.loop(0, ng_self, step=2)
    def _(jg):
        _group(jg, s0, d0, s1, d1, 0)
        _group(jg + 1, s1, d1, s0, d0, G % NBUF)

    for i in range(P):
        _wait_gather(i % NBUF)
    plsc.subcore_barrier()
    pltpu.sync_copy(acc.at[pl.ds(s * ROWS_PER_SUB, ROWS_PER_SUB)],
                    out_hbm.at[c, pl.ds(s * ROWS_PER_SUB, ROWS_PER_SUB)])


def _make_deg_kernel(interpret=False):
    return pl.kernel(
        _deg_body,
        out_type=jax.ShapeDtypeStruct((NCORES, NPAD, DW), jnp.float32),
        mesh=_vector_mesh,
        scratch_types=[
            pltpu.VMEM_SHARED((NPAD, DW), jnp.float32),
            pltpu.VMEM((G, K), jnp.int32),
            pltpu.VMEM((K, DW), jnp.float32),
            pltpu.VMEM((K, DW), jnp.float32),
        ],
        interpret=interpret,
    )


def _make_segsum_kernel(interpret=False):
    return pl.kernel(
        _segsum_body,
        out_type=jax.ShapeDtypeStruct((NCORES, NPAD, D), jnp.float32),
        mesh=_vector_mesh,
        scratch_types=(
            [pltpu.VMEM_SHARED((NPAD, D), jnp.float32)]
            + [pltpu.VMEM((G, K), jnp.int32)] * 4
            + [pltpu.VMEM((K, D), jnp.float32)] * NBUF
            + [pltpu.SemaphoreType.DMA] * (NBUF + 1)
        ),
        interpret=interpret,
    )


_deg_kernel = _make_deg_kernel()
_segsum_kernel = _make_segsum_kernel()


def _dinv_block(dcol):
    return jnp.where(dcol > 0.0, lax.rsqrt(jnp.maximum(dcol, 1e-30)), 0.0)


def _mm_first_body(x_ref, w_ref, dcol_ref, g_ref):
    dv = _dinv_block(dcol_ref[...])
    h = jnp.dot(x_ref[...], w_ref[...], preferred_element_type=jnp.float32)
    g_ref[...] = h * dv


def _mm_mid_body(accs_ref, dcol_ref, b_ref, w_ref, g_ref):
    dv = _dinv_block(dcol_ref[...])
    act = jnp.maximum(dv * (accs_ref[0] + accs_ref[1]) + b_ref[...], 0.0)
    g_ref[...] = jnp.dot(act, w_ref[...], preferred_element_type=jnp.float32) * dv


def _epilogue_body(accs_ref, dcol_ref, b_ref, out_ref):
    dv = _dinv_block(dcol_ref[...])
    out_ref[...] = jnp.maximum(dv * (accs_ref[0] + accs_ref[1]) + b_ref[...], 0.0)


_GRID = (NPAD // RB,)
_accs_spec = pl.BlockSpec((NCORES, RB, D), lambda i: (0, i, 0))
_dcol_spec = pl.BlockSpec((RB, 1), lambda i: (i, 0))
_b_spec = pl.BlockSpec((1, D), lambda i: (0, 0))
_w_spec = pl.BlockSpec((D, D), lambda i: (0, 0))
_row_spec = pl.BlockSpec((RB, D), lambda i: (i, 0))

_mm_first = pl.pallas_call(
    _mm_first_body,
    grid=_GRID,
    in_specs=[_row_spec, _w_spec, _dcol_spec],
    out_specs=_row_spec,
    out_shape=jax.ShapeDtypeStruct((NPAD, D), jnp.float32),
)

_mm_mid = pl.pallas_call(
    _mm_mid_body,
    grid=_GRID,
    in_specs=[_accs_spec, _dcol_spec, _b_spec, _w_spec],
    out_specs=_row_spec,
    out_shape=jax.ShapeDtypeStruct((NPAD, D), jnp.float32),
)

_epilogue = pl.pallas_call(
    _epilogue_body,
    grid=_GRID,
    in_specs=[_accs_spec, _dcol_spec, _b_spec],
    out_specs=_row_spec,
    out_shape=jax.ShapeDtypeStruct((NPAD, D), jnp.float32),
)


def _pipeline(src2d, dst2d, xp, W1, b1r, W2, b2r, W3, b3r):
    src2d = src2d[0]
    dst2d = dst2d[0]
    deg_parts = _deg_kernel(dst2d)
    dcol = lax.psum(deg_parts[0, :, 0:1] + deg_parts[1, :, 0:1], "dev")

    g = _mm_first(xp, W1, dcol)
    accs = lax.psum(_segsum_kernel(g, src2d, dst2d), "dev")
    g = _mm_mid(accs, dcol, b1r, W2)
    accs = lax.psum(_segsum_kernel(g, src2d, dst2d), "dev")
    g = _mm_mid(accs, dcol, b2r, W3)
    accs = lax.psum(_segsum_kernel(g, src2d, dst2d), "dev")
    return _epilogue(accs, dcol, b3r)


def kernel(x, edge_index, W1, b1, W2, b2, W3, b3):
    loop = jnp.arange(N_NODES, dtype=jnp.int32)
    src = jnp.concatenate([edge_index[0].astype(jnp.int32), loop])
    dst = jnp.concatenate([edge_index[1].astype(jnp.int32), loop])
    pad = jnp.full((EPAD - E_DEV,), DUMMY, dtype=jnp.int32)

    def build(v):
        parts = []
        for d in range(NDEV):
            vd = jnp.concatenate([v[d * E_DEV:(d + 1) * E_DEV], pad])
            ea = vd[:NSUB * CAP_A].reshape(NSUB, NGA, G, K)
            eb = vd[NSUB * CAP_A:].reshape(NSUB, NGB, G, K)
            da = jnp.full((NSUB, NGMAX + 1 - NGA, G, K), DUMMY, jnp.int32)
            db = jnp.full((NSUB, NGMAX + 1 - NGB, G, K), DUMMY, jnp.int32)
            parts.append(jnp.concatenate([
                jnp.concatenate([ea, da], axis=1),
                jnp.concatenate([eb, db], axis=1)], axis=0))
        return jnp.stack(parts)

    src2d = build(src)
    dst2d = build(dst)

    xp = jnp.pad(x, ((0, NPAD - N_NODES), (0, 0)))
    b1r = b1.reshape(1, D)
    b2r = b2.reshape(1, D)
    b3r = b3.reshape(1, D)

    import numpy as _np
    from jax.sharding import Mesh, PartitionSpec as PS
    mesh = Mesh(_np.asarray(jax.devices()[:NDEV]), ("dev",))
    rep = PS()
    shfn = jax.shard_map(
        _pipeline, mesh=mesh,
        in_specs=(PS("dev"), PS("dev"), rep, rep, rep, rep, rep, rep, rep),
        out_specs=rep, check_vma=False)
    out = shfn(src2d, dst2d, xp, W1, b1r, W2, b2r, W3, b3r)
    return out[:N_NODES]
